# Initial kernel scaffold; baseline (speedup 1.0000x reference)
import functools

import jax
import jax.numpy as jnp
from jax import lax
from jax.experimental import pallas as pl
from jax.experimental.pallas import tpu as pltpu
from jax.experimental.pallas import tpu_sc as plsc

N = 50000
E = 800000
NUM_SPECIES = 4
NUM_BASIS = 8
MUL = 16
NLAYER = 3
HIDDEN = 64
NUM_GRAPHS = 64
R_MAX = 5.0
NORM = 0.25

_B = 128
_NCHUNK = E // _B
_NW = 32
_HALF = N // 2
_HALF_PAD = 25088
_RPT = _HALF_PAD // 16
_ZROWS = 224

_EB = 256
_NB = 2000


@functools.cache
def _make_gather(D):
    mesh = plsc.VectorSubcoreMesh(core_axis_name="c", subcore_axis_name="s")
    cpw = -(-_NCHUNK // _NW)

    @functools.partial(
        pl.kernel, mesh=mesh,
        out_type=jax.ShapeDtypeStruct((E, D), jnp.float32),
        scratch_types=[
            pltpu.VMEM((_B,), jnp.int32),
            pltpu.VMEM((_B, D), jnp.float32),
            pltpu.SemaphoreType.DMA,
        ],
    )
    def gather(table_hbm, idx_hbm, out_hbm, idx_v, rows_v, sem):
        c = lax.axis_index("c")
        s = lax.axis_index("s")
        w = s * 2 + c
        start = w * cpw
        nthis = jnp.clip(_NCHUNK - start, 0, cpw)

        def body(g, carry):
            base = (start + g) * _B
            pltpu.sync_copy(idx_hbm.at[pl.ds(base, _B)], idx_v)
            pltpu.async_copy(table_hbm.at[idx_v], rows_v, sem).wait()
            pltpu.sync_copy(rows_v, out_hbm.at[pl.ds(base, _B)])
            return carry

        lax.fori_loop(0, nthis, body, 0)

    return gather


@functools.cache
def _make_scatter(D):
    mesh = plsc.VectorSubcoreMesh(core_axis_name="c", subcore_axis_name="s")
    cps = -(-_NCHUNK // 16)

    @functools.partial(
        pl.kernel, mesh=mesh,
        out_type=jax.ShapeDtypeStruct((N, D), jnp.float32),
        scratch_types=[
            pltpu.VMEM((_B,), jnp.int32),
            pltpu.VMEM((_B,), jnp.int32),
            pltpu.VMEM((_B, D), jnp.float32),
            pltpu.VMEM((_ZROWS, D), jnp.float32),
            pltpu.VMEM_SHARED((_HALF_PAD, D), jnp.float32),
            pltpu.SemaphoreType.DMA,
        ],
    )
    def scatter(msgs_hbm, dst_hbm, zeros_hbm, out_hbm,
                didx_v, lidx_v, rows_v, z_v, acc_sh, sem):
        c = lax.axis_index("c")
        s = lax.axis_index("s")
        lo = c * _HALF
        pltpu.sync_copy(zeros_hbm, z_v)
        for z in range(_RPT // _ZROWS):
            pltpu.sync_copy(
                z_v, acc_sh.at[pl.ds(s * _RPT + z * _ZROWS, _ZROWS)])
        plsc.subcore_barrier()

        start = s * cps
        nthis = jnp.clip(_NCHUNK - start, 0, cps)

        def body(g, carry):
            base = (start + g) * _B
            pltpu.sync_copy(dst_hbm.at[pl.ds(base, _B)], didx_v)
            pltpu.sync_copy(msgs_hbm.at[pl.ds(base, _B)], rows_v)
            for k in range(_B // 16):
                d16 = didx_v[pl.ds(k * 16, 16)]
                loc = d16 - lo
                ok = (loc >= 0) & (loc < _HALF)
                lidx_v[pl.ds(k * 16, 16)] = jnp.where(ok, loc, _HALF)
            pltpu.sync_copy(rows_v, acc_sh.at[lidx_v], add=True)
            return carry

        lax.fori_loop(0, nthis, body, 0)
        plsc.subcore_barrier()

        rstart = s * _RPT
        tail = _HALF - 15 * _RPT

        @pl.when(s < 15)
        def _():
            pltpu.sync_copy(acc_sh.at[pl.ds(rstart, _RPT)],
                            out_hbm.at[pl.ds(lo + rstart, _RPT)])

        @pl.when(s == 15)
        def _():
            pltpu.sync_copy(acc_sh.at[pl.ds(rstart, tail)],
                            out_hbm.at[pl.ds(lo + rstart, tail)])

    return scatter


def _dot(a, b):
    return jnp.dot(a, b, preferred_element_type=jnp.float32)


def _sigmoid(x):
    return 1.0 / (1.0 + jnp.exp(-x))


def _tc_geom(ps, pd):
    def body(ps_ref, pd_ref, o_ref):
        a = ps_ref[...]
        b = pd_ref[...]
        vec = b[:, 0:3] - a[:, 0:3]
        r = jnp.sqrt(jnp.sum(vec * vec, axis=1, keepdims=True)) + 1e-9
        y = vec / r
        x = r * (1.0 / R_MAX)
        x2 = x * x
        x3 = x2 * x
        x6 = x3 * x3
        env = 1.0 - 28.0 * x6 + 48.0 * x6 * x - 21.0 * x6 * x2
        env = jnp.where(x < 1.0, env, 0.0)
        nvec = lax.broadcasted_iota(jnp.float32, (1, NUM_BASIS), 1) + 1.0
        arg = nvec * ((jnp.pi / R_MAX) * r)
        basis = (jnp.sqrt(2.0 / R_MAX) * env / r) * jnp.sin(arg)
        o_ref[...] = jnp.concatenate(
            [basis, y, jnp.zeros((_EB, 5), jnp.float32)], axis=1)

    return pl.pallas_call(
        body, grid=(E // _EB,),
        in_specs=[pl.BlockSpec((_EB, 8), lambda i: (i, 0)),
                  pl.BlockSpec((_EB, 8), lambda i: (i, 0))],
        out_specs=pl.BlockSpec((_EB, 16), lambda i: (i, 0)),
        out_shape=jax.ShapeDtypeStruct((E, 16), jnp.float32),
    )(ps, pd)


def _tc_edge(geom, G, W1l, W2l):
    def body(geom_ref, g_ref, w1_ref, w2_ref, m_ref):
        ge = geom_ref[...]
        basis = ge[:, 0:8]
        yx = ge[:, 8:9]
        yy = ge[:, 9:10]
        yz = ge[:, 10:11]
        h = _dot(basis, w1_ref[...])
        h = h * _sigmoid(h)
        w = _dot(h, w2_ref[...])
        w0 = w[:, 0:16]
        w1 = w[:, 16:32]
        w2 = w[:, 32:48]
        w3 = w[:, 48:64]
        w4 = w[:, 64:80]
        sj = g_ref[:, 0:16]
        vx = g_ref[:, 16:32]
        vy = g_ref[:, 32:48]
        vz = g_ref[:, 48:64]
        dvy = vx * yx + vy * yy + vz * yz
        m_s = w0 * sj + w3 * dvy
        t = w1 * sj
        cx = vy * yz - vz * yy
        cy = vz * yx - vx * yz
        cz = vx * yy - vy * yx
        mx = t * yx + w2 * vx + w4 * cx
        my = t * yy + w2 * vy + w4 * cy
        mz = t * yz + w2 * vz + w4 * cz
        m_ref[...] = jnp.concatenate([m_s, mx, my, mz], axis=1)

    return pl.pallas_call(
        body, grid=(E // _EB,),
        in_specs=[pl.BlockSpec((_EB, 16), lambda i: (i, 0)),
                  pl.BlockSpec((_EB, 64), lambda i: (i, 0)),
                  pl.BlockSpec((NUM_BASIS, HIDDEN), lambda i: (0, 0)),
                  pl.BlockSpec((HIDDEN, 80), lambda i: (0, 0))],
        out_specs=pl.BlockSpec((_EB, 64), lambda i: (i, 0)),
        out_shape=jax.ShapeDtypeStruct((E, 64), jnp.float32),
    )(geom, G, W1l, W2l)


def _tc_edge_s(geom, G, W1l, W2s):
    def body(geom_ref, g_ref, w1_ref, w2_ref, m_ref):
        ge = geom_ref[...]
        basis = ge[:, 0:8]
        yx = ge[:, 8:9]
        yy = ge[:, 9:10]
        yz = ge[:, 10:11]
        h = _dot(basis, w1_ref[...])
        h = h * _sigmoid(h)
        w = _dot(h, w2_ref[...])
        sj = g_ref[:, 0:16]
        vx = g_ref[:, 16:32]
        vy = g_ref[:, 32:48]
        vz = g_ref[:, 48:64]
        dvy = vx * yx + vy * yy + vz * yz
        m_ref[...] = w[:, 0:16] * sj + w[:, 16:32] * dvy

    return pl.pallas_call(
        body, grid=(E // _EB,),
        in_specs=[pl.BlockSpec((_EB, 16), lambda i: (i, 0)),
                  pl.BlockSpec((_EB, 64), lambda i: (i, 0)),
                  pl.BlockSpec((NUM_BASIS, HIDDEN), lambda i: (0, 0)),
                  pl.BlockSpec((HIDDEN, 32), lambda i: (0, 0))],
        out_specs=pl.BlockSpec((_EB, 16), lambda i: (i, 0)),
        out_shape=jax.ShapeDtypeStruct((E, 16), jnp.float32),
    )(geom, G, W1l, W2s)


def _tc_init(sp1, W_chem, Ws10):
    def body(sp_ref, wc_ref, w_ref, f_ref, f1_ref):
        sp = sp_ref[...]
        oh = (sp == lax.broadcasted_iota(
            jnp.float32, (_NB, NUM_SPECIES), 1)).astype(jnp.float32)
        s0 = _dot(oh, wc_ref[...])
        z = jnp.zeros((_NB, 48), jnp.float32)
        f_ref[...] = jnp.concatenate([s0, z], axis=1)
        f1_ref[...] = jnp.concatenate([_dot(s0, w_ref[...]), z], axis=1)

    return pl.pallas_call(
        body, grid=(N // _NB,),
        in_specs=[pl.BlockSpec((_NB, 1), lambda i: (i, 0)),
                  pl.BlockSpec((NUM_SPECIES, MUL), lambda i: (0, 0)),
                  pl.BlockSpec((MUL, MUL), lambda i: (0, 0))],
        out_specs=[pl.BlockSpec((_NB, 64), lambda i: (i, 0)),
                   pl.BlockSpec((_NB, 64), lambda i: (i, 0))],
        out_shape=[jax.ShapeDtypeStruct((N, 64), jnp.float32),
                   jax.ShapeDtypeStruct((N, 64), jnp.float32)],
    )(sp1, W_chem, Ws10)


def _tc_post(A, Fin, sp1, Ws2l, Wv2l, Wscsl, Wscvl, Wgl, Ws1n, Wv1n):
    def body(a_ref, fin_ref, sp_ref, ws2_ref, wv2_ref, wss_ref, wsv_ref,
             wg_ref, ws1_ref, wv1_ref, fout_ref, f1_ref):
        a = a_ref[...]
        fin = fin_ref[...]
        sp = sp_ref[...]
        oh = (sp == lax.broadcasted_iota(
            jnp.float32, (_NB, NUM_SPECIES), 1)).astype(jnp.float32)
        s_in = fin[:, 0:16]
        vix = fin[:, 16:32]
        viy = fin[:, 32:48]
        viz = fin[:, 48:64]
        s2 = _dot(a[:, 0:16] * NORM, ws2_ref[...])
        wv2 = wv2_ref[...]
        v2x = _dot(a[:, 16:32] * NORM, wv2)
        v2y = _dot(a[:, 32:48] * NORM, wv2)
        v2z = _dot(a[:, 48:64] * NORM, wv2)
        for spc in range(NUM_SPECIES):
            m = oh[:, spc:spc + 1]
            s2 = s2 + m * _dot(s_in, wss_ref[spc])
            wv = wsv_ref[spc]
            v2x = v2x + m * _dot(vix, wv)
            v2y = v2y + m * _dot(viy, wv)
            v2z = v2z + m * _dot(viz, wv)
        g = _sigmoid(_dot(s2, wg_ref[...]))
        s_new = s2 * _sigmoid(s2) + s_in
        vnx = v2x * g + vix
        vny = v2y * g + viy
        vnz = v2z * g + viz
        fout_ref[...] = jnp.concatenate([s_new, vnx, vny, vnz], axis=1)
        ws1 = ws1_ref[...]
        wv1 = wv1_ref[...]
        f1_ref[...] = jnp.concatenate(
            [_dot(s_new, ws1), _dot(vnx, wv1),
             _dot(vny, wv1), _dot(vnz, wv1)], axis=1)

    return pl.pallas_call(
        body, grid=(N // _NB,),
        in_specs=[pl.BlockSpec((_NB, 64), lambda i: (i, 0)),
                  pl.BlockSpec((_NB, 64), lambda i: (i, 0)),
                  pl.BlockSpec((_NB, 1), lambda i: (i, 0)),
                  pl.BlockSpec((MUL, MUL), lambda i: (0, 0)),
                  pl.BlockSpec((MUL, MUL), lambda i: (0, 0)),
                  pl.BlockSpec((NUM_SPECIES, MUL, MUL), lambda i: (0, 0, 0)),
                  pl.BlockSpec((NUM_SPECIES, MUL, MUL), lambda i: (0, 0, 0)),
                  pl.BlockSpec((MUL, MUL), lambda i: (0, 0)),
                  pl.BlockSpec((MUL, MUL), lambda i: (0, 0)),
                  pl.BlockSpec((MUL, MUL), lambda i: (0, 0))],
        out_specs=[pl.BlockSpec((_NB, 64), lambda i: (i, 0)),
                   pl.BlockSpec((_NB, 64), lambda i: (i, 0))],
        out_shape=[jax.ShapeDtypeStruct((N, 64), jnp.float32),
                   jax.ShapeDtypeStruct((N, 64), jnp.float32)],
    )(A, Fin, sp1, Ws2l, Wv2l, Wscsl, Wscvl, Wgl, Ws1n, Wv1n)


def _tc_head(A, Fin, sp1, bid1, Ws2l, Wscsl, W_oh, W_out):
    def body(a_ref, fin_ref, sp_ref, bid_ref, ws2_ref, wss_ref,
             woh_ref, wout_ref, o_ref):
        i = pl.program_id(0)
        s_in = fin_ref[:, 0:16]
        sp = sp_ref[...]
        oh = (sp == lax.broadcasted_iota(
            jnp.float32, (_NB, NUM_SPECIES), 1)).astype(jnp.float32)
        s2 = _dot(a_ref[...] * NORM, ws2_ref[...])
        for spc in range(NUM_SPECIES):
            s2 = s2 + oh[:, spc:spc + 1] * _dot(s_in, wss_ref[spc])
        s_new = s2 * _sigmoid(s2) + s_in
        e = _dot(_dot(s_new, woh_ref[...]), wout_ref[...])
        ohb = (bid_ref[...] == lax.broadcasted_iota(
            jnp.float32, (_NB, NUM_GRAPHS), 1)).astype(jnp.float32)
        contrib = jnp.sum(ohb * e, axis=0, keepdims=True)

        @pl.when(i == 0)
        def _():
            o_ref[...] = jnp.zeros_like(o_ref)

        o_ref[...] += contrib

    out = pl.pallas_call(
        body, grid=(N // _NB,),
        in_specs=[pl.BlockSpec((_NB, 16), lambda i: (i, 0)),
                  pl.BlockSpec((_NB, 64), lambda i: (i, 0)),
                  pl.BlockSpec((_NB, 1), lambda i: (i, 0)),
                  pl.BlockSpec((_NB, 1), lambda i: (i, 0)),
                  pl.BlockSpec((MUL, MUL), lambda i: (0, 0)),
                  pl.BlockSpec((NUM_SPECIES, MUL, MUL), lambda i: (0, 0, 0)),
                  pl.BlockSpec((MUL, MUL), lambda i: (0, 0)),
                  pl.BlockSpec((MUL, 1), lambda i: (0, 0))],
        out_specs=pl.BlockSpec((1, NUM_GRAPHS), lambda i: (0, 0)),
        out_shape=jax.ShapeDtypeStruct((1, NUM_GRAPHS), jnp.float32),
    )(A, Fin, sp1, bid1, Ws2l, Wscsl, W_oh, W_out)
    return out


def kernel(pos, W_chem, W_rad1, W_rad2, Ws1, Wv1, Ws2, Wv2,
           Wsc_s, Wsc_v, Wg, W_oh, W_out, edge_index, species, batch_ids):
    f32 = jnp.float32
    ei = jnp.asarray(edge_index, jnp.int32)
    srci = ei[0]
    dsti = ei[1]
    sp1 = jnp.asarray(species, f32).reshape(N, 1)
    bid1 = jnp.asarray(batch_ids, f32).reshape(N, 1)
    pos8 = jnp.concatenate([pos, jnp.zeros((N, 5), f32)], axis=1)
    z64 = jnp.zeros((_ZROWS, 64), f32)
    z16 = jnp.zeros((_ZROWS, 16), f32)

    gather8 = _make_gather(8)
    gather64 = _make_gather(64)
    scatter64 = _make_scatter(64)
    scatter16 = _make_scatter(16)

    ps = gather8(pos8, srci)
    pd = gather8(pos8, dsti)
    geom = _tc_geom(ps, pd)

    F, F1 = _tc_init(sp1, W_chem, Ws1[0])
    for l in range(NLAYER):
        G = gather64(F1, srci)
        if l < NLAYER - 1:
            M = _tc_edge(geom, G, W_rad1[l], W_rad2[l])
            A = scatter64(M, dsti, z64)
            F, F1 = _tc_post(A, F, sp1, Ws2[l], Wv2[l], Wsc_s[l],
                             Wsc_v[l], Wg[l], Ws1[l + 1], Wv1[l + 1])
        else:
            W2s = jnp.concatenate(
                [W_rad2[l][:, 0:16], W_rad2[l][:, 48:64]], axis=1)
            M = _tc_edge_s(geom, G, W_rad1[l], W2s)
            A = scatter16(M, dsti, z16)
            out = _tc_head(A, F, sp1, bid1, Ws2[l], Wsc_s[l], W_oh, W_out)
    return out.reshape(NUM_GRAPHS, 1)


# speedup vs baseline: 10.9359x; 10.9359x over previous
import functools

import jax
import jax.numpy as jnp
from jax import lax
from jax.experimental import pallas as pl
from jax.experimental.pallas import tpu as pltpu
from jax.experimental.pallas import tpu_sc as plsc

N = 50000
E = 800000
NUM_SPECIES = 4
NUM_BASIS = 8
MUL = 16
NLAYER = 3
HIDDEN = 64
NUM_GRAPHS = 64
R_MAX = 5.0
NORM = 0.25

_B = 128
_NCHUNK = E // _B
_NW = 32
_HALF = N // 2
_HALF_PAD = 25088
_RPT = _HALF_PAD // 16
_ZROWS = 224

_EB = 256
_NB = 2000


@functools.cache
def _make_gather(D):
    mesh = plsc.VectorSubcoreMesh(core_axis_name="c", subcore_axis_name="s")
    cpw = -(-_NCHUNK // _NW)

    @functools.partial(
        pl.kernel, mesh=mesh,
        compiler_params=pltpu.CompilerParams(use_tc_tiling_on_sc=False),
        out_type=jax.ShapeDtypeStruct((E, D), jnp.float32),
        scratch_types=[
            pltpu.VMEM((_B,), jnp.int32),
            pltpu.VMEM((_B, D), jnp.float32),
            pltpu.SemaphoreType.DMA,
        ],
    )
    def gather(table_hbm, idx_hbm, out_hbm, idx_v, rows_v, sem):
        c = lax.axis_index("c")
        s = lax.axis_index("s")
        w = s * 2 + c
        start = w * cpw
        nthis = jnp.clip(_NCHUNK - start, 0, cpw)

        def body(g, carry):
            base = (start + g) * _B
            pltpu.sync_copy(idx_hbm.at[pl.ds(base, _B)], idx_v)
            pltpu.async_copy(table_hbm.at[idx_v], rows_v, sem).wait()
            pltpu.sync_copy(rows_v, out_hbm.at[pl.ds(base, _B)])
            return carry

        lax.fori_loop(0, nthis, body, 0)

    return gather


@functools.cache
def _make_scatter(D):
    mesh = plsc.VectorSubcoreMesh(core_axis_name="c", subcore_axis_name="s")
    cps = -(-_NCHUNK // 16)

    @functools.partial(
        pl.kernel, mesh=mesh,
        compiler_params=pltpu.CompilerParams(use_tc_tiling_on_sc=False),
        out_type=jax.ShapeDtypeStruct((N, D), jnp.float32),
        scratch_types=[
            pltpu.VMEM((_B,), jnp.int32),
            pltpu.VMEM((_B,), jnp.int32),
            pltpu.VMEM((_B, D), jnp.float32),
            pltpu.VMEM((_ZROWS, D), jnp.float32),
            pltpu.VMEM_SHARED((_HALF_PAD, D), jnp.float32),
            pltpu.SemaphoreType.DMA,
        ],
    )
    def scatter(msgs_hbm, dst_hbm, zeros_hbm, out_hbm,
                didx_v, lidx_v, rows_v, z_v, acc_sh, sem):
        c = lax.axis_index("c")
        s = lax.axis_index("s")
        lo = c * _HALF
        pltpu.sync_copy(zeros_hbm, z_v)
        for z in range(_RPT // _ZROWS):
            pltpu.sync_copy(
                z_v, acc_sh.at[pl.ds(s * _RPT + z * _ZROWS, _ZROWS)])
        plsc.subcore_barrier()

        start = s * cps
        nthis = jnp.clip(_NCHUNK - start, 0, cps)

        def body(g, carry):
            base = (start + g) * _B
            pltpu.sync_copy(dst_hbm.at[pl.ds(base, _B)], didx_v)
            pltpu.sync_copy(msgs_hbm.at[pl.ds(base, _B)], rows_v)
            for k in range(_B // 16):
                d16 = didx_v[pl.ds(k * 16, 16)]
                loc = d16 - lo
                ok = (loc >= 0) & (loc < _HALF)
                lidx_v[pl.ds(k * 16, 16)] = jnp.where(ok, loc, _HALF)
            pltpu.sync_copy(rows_v, acc_sh.at[lidx_v], add=True)
            return carry

        lax.fori_loop(0, nthis, body, 0)
        plsc.subcore_barrier()

        rstart = s * _RPT
        tail = _HALF - 15 * _RPT

        @pl.when(s < 15)
        def _():
            pltpu.sync_copy(acc_sh.at[pl.ds(rstart, _RPT)],
                            out_hbm.at[pl.ds(lo + rstart, _RPT)])

        @pl.when(s == 15)
        def _():
            pltpu.sync_copy(acc_sh.at[pl.ds(rstart, tail)],
                            out_hbm.at[pl.ds(lo + rstart, tail)])

    return scatter


def _dot(a, b):
    return jnp.dot(a, b, preferred_element_type=jnp.float32)


def _sigmoid(x):
    return 1.0 / (1.0 + jnp.exp(-x))


def _tc_geom(ps, pd):
    def body(ps_ref, pd_ref, o_ref):
        a = ps_ref[...]
        b = pd_ref[...]
        vec = b[:, 0:3] - a[:, 0:3]
        r = jnp.sqrt(jnp.sum(vec * vec, axis=1, keepdims=True)) + 1e-9
        y = vec / r
        x = r * (1.0 / R_MAX)
        x2 = x * x
        x3 = x2 * x
        x6 = x3 * x3
        env = 1.0 - 28.0 * x6 + 48.0 * x6 * x - 21.0 * x6 * x2
        env = jnp.where(x < 1.0, env, 0.0)
        nvec = lax.broadcasted_iota(
            jnp.int32, (1, NUM_BASIS), 1).astype(jnp.float32) + 1.0
        arg = nvec * ((jnp.pi / R_MAX) * r)
        basis = (jnp.sqrt(2.0 / R_MAX) * env / r) * jnp.sin(arg)
        o_ref[...] = jnp.concatenate(
            [basis, y, jnp.zeros((_EB, 5), jnp.float32)], axis=1)

    return pl.pallas_call(
        body, grid=(E // _EB,),
        in_specs=[pl.BlockSpec((_EB, 8), lambda i: (i, 0)),
                  pl.BlockSpec((_EB, 8), lambda i: (i, 0))],
        out_specs=pl.BlockSpec((_EB, 16), lambda i: (i, 0)),
        out_shape=jax.ShapeDtypeStruct((E, 16), jnp.float32),
    )(ps, pd)


def _tc_edge(geom, G, W1l, W2l):
    def body(geom_ref, g_ref, w1_ref, w2_ref, m_ref):
        ge = geom_ref[...]
        basis = ge[:, 0:8]
        yx = ge[:, 8:9]
        yy = ge[:, 9:10]
        yz = ge[:, 10:11]
        h = _dot(basis, w1_ref[...])
        h = h * _sigmoid(h)
        w = _dot(h, w2_ref[...])
        w0 = w[:, 0:16]
        w1 = w[:, 16:32]
        w2 = w[:, 32:48]
        w3 = w[:, 48:64]
        w4 = w[:, 64:80]
        sj = g_ref[:, 0:16]
        vx = g_ref[:, 16:32]
        vy = g_ref[:, 32:48]
        vz = g_ref[:, 48:64]
        dvy = vx * yx + vy * yy + vz * yz
        m_s = w0 * sj + w3 * dvy
        t = w1 * sj
        cx = vy * yz - vz * yy
        cy = vz * yx - vx * yz
        cz = vx * yy - vy * yx
        mx = t * yx + w2 * vx + w4 * cx
        my = t * yy + w2 * vy + w4 * cy
        mz = t * yz + w2 * vz + w4 * cz
        m_ref[...] = jnp.concatenate([m_s, mx, my, mz], axis=1)

    return pl.pallas_call(
        body, grid=(E // _EB,),
        in_specs=[pl.BlockSpec((_EB, 16), lambda i: (i, 0)),
                  pl.BlockSpec((_EB, 64), lambda i: (i, 0)),
                  pl.BlockSpec((NUM_BASIS, HIDDEN), lambda i: (0, 0)),
                  pl.BlockSpec((HIDDEN, 80), lambda i: (0, 0))],
        out_specs=pl.BlockSpec((_EB, 64), lambda i: (i, 0)),
        out_shape=jax.ShapeDtypeStruct((E, 64), jnp.float32),
    )(geom, G, W1l, W2l)


def _tc_edge_s(geom, G, W1l, W2s):
    def body(geom_ref, g_ref, w1_ref, w2_ref, m_ref):
        ge = geom_ref[...]
        basis = ge[:, 0:8]
        yx = ge[:, 8:9]
        yy = ge[:, 9:10]
        yz = ge[:, 10:11]
        h = _dot(basis, w1_ref[...])
        h = h * _sigmoid(h)
        w = _dot(h, w2_ref[...])
        sj = g_ref[:, 0:16]
        vx = g_ref[:, 16:32]
        vy = g_ref[:, 32:48]
        vz = g_ref[:, 48:64]
        dvy = vx * yx + vy * yy + vz * yz
        m_ref[...] = w[:, 0:16] * sj + w[:, 16:32] * dvy

    return pl.pallas_call(
        body, grid=(E // _EB,),
        in_specs=[pl.BlockSpec((_EB, 16), lambda i: (i, 0)),
                  pl.BlockSpec((_EB, 64), lambda i: (i, 0)),
                  pl.BlockSpec((NUM_BASIS, HIDDEN), lambda i: (0, 0)),
                  pl.BlockSpec((HIDDEN, 32), lambda i: (0, 0))],
        out_specs=pl.BlockSpec((_EB, 16), lambda i: (i, 0)),
        out_shape=jax.ShapeDtypeStruct((E, 16), jnp.float32),
    )(geom, G, W1l, W2s)


def _tc_init(sp1, W_chem, Ws10):
    def body(sp_ref, wc_ref, w_ref, f_ref, f1_ref):
        sp = sp_ref[...]
        oh = (sp == lax.broadcasted_iota(
            jnp.int32, (_NB, NUM_SPECIES), 1).astype(jnp.float32)
              ).astype(jnp.float32)
        s0 = _dot(oh, wc_ref[...])
        z = jnp.zeros((_NB, 48), jnp.float32)
        f_ref[...] = jnp.concatenate([s0, z], axis=1)
        f1_ref[...] = jnp.concatenate([_dot(s0, w_ref[...]), z], axis=1)

    return pl.pallas_call(
        body, grid=(N // _NB,),
        in_specs=[pl.BlockSpec((_NB, 1), lambda i: (i, 0)),
                  pl.BlockSpec((NUM_SPECIES, MUL), lambda i: (0, 0)),
                  pl.BlockSpec((MUL, MUL), lambda i: (0, 0))],
        out_specs=[pl.BlockSpec((_NB, 64), lambda i: (i, 0)),
                   pl.BlockSpec((_NB, 64), lambda i: (i, 0))],
        out_shape=[jax.ShapeDtypeStruct((N, 64), jnp.float32),
                   jax.ShapeDtypeStruct((N, 64), jnp.float32)],
    )(sp1, W_chem, Ws10)


def _tc_post(A, Fin, sp1, Ws2l, Wv2l, Wscsl, Wscvl, Wgl, Ws1n, Wv1n):
    def body(a_ref, fin_ref, sp_ref, ws2_ref, wv2_ref, wss_ref, wsv_ref,
             wg_ref, ws1_ref, wv1_ref, fout_ref, f1_ref):
        a = a_ref[...]
        fin = fin_ref[...]
        sp = sp_ref[...]
        oh = (sp == lax.broadcasted_iota(
            jnp.int32, (_NB, NUM_SPECIES), 1).astype(jnp.float32)
              ).astype(jnp.float32)
        s_in = fin[:, 0:16]
        vix = fin[:, 16:32]
        viy = fin[:, 32:48]
        viz = fin[:, 48:64]
        s2 = _dot(a[:, 0:16] * NORM, ws2_ref[...])
        wv2 = wv2_ref[...]
        v2x = _dot(a[:, 16:32] * NORM, wv2)
        v2y = _dot(a[:, 32:48] * NORM, wv2)
        v2z = _dot(a[:, 48:64] * NORM, wv2)
        for spc in range(NUM_SPECIES):
            m = oh[:, spc:spc + 1]
            s2 = s2 + m * _dot(s_in, wss_ref[spc])
            wv = wsv_ref[spc]
            v2x = v2x + m * _dot(vix, wv)
            v2y = v2y + m * _dot(viy, wv)
            v2z = v2z + m * _dot(viz, wv)
        g = _sigmoid(_dot(s2, wg_ref[...]))
        s_new = s2 * _sigmoid(s2) + s_in
        vnx = v2x * g + vix
        vny = v2y * g + viy
        vnz = v2z * g + viz
        fout_ref[...] = jnp.concatenate([s_new, vnx, vny, vnz], axis=1)
        ws1 = ws1_ref[...]
        wv1 = wv1_ref[...]
        f1_ref[...] = jnp.concatenate(
            [_dot(s_new, ws1), _dot(vnx, wv1),
             _dot(vny, wv1), _dot(vnz, wv1)], axis=1)

    return pl.pallas_call(
        body, grid=(N // _NB,),
        in_specs=[pl.BlockSpec((_NB, 64), lambda i: (i, 0)),
                  pl.BlockSpec((_NB, 64), lambda i: (i, 0)),
                  pl.BlockSpec((_NB, 1), lambda i: (i, 0)),
                  pl.BlockSpec((MUL, MUL), lambda i: (0, 0)),
                  pl.BlockSpec((MUL, MUL), lambda i: (0, 0)),
                  pl.BlockSpec((NUM_SPECIES, MUL, MUL), lambda i: (0, 0, 0)),
                  pl.BlockSpec((NUM_SPECIES, MUL, MUL), lambda i: (0, 0, 0)),
                  pl.BlockSpec((MUL, MUL), lambda i: (0, 0)),
                  pl.BlockSpec((MUL, MUL), lambda i: (0, 0)),
                  pl.BlockSpec((MUL, MUL), lambda i: (0, 0))],
        out_specs=[pl.BlockSpec((_NB, 64), lambda i: (i, 0)),
                   pl.BlockSpec((_NB, 64), lambda i: (i, 0))],
        out_shape=[jax.ShapeDtypeStruct((N, 64), jnp.float32),
                   jax.ShapeDtypeStruct((N, 64), jnp.float32)],
    )(A, Fin, sp1, Ws2l, Wv2l, Wscsl, Wscvl, Wgl, Ws1n, Wv1n)


def _tc_head(A, Fin, sp1, bid1, Ws2l, Wscsl, W_oh, W_out):
    def body(a_ref, fin_ref, sp_ref, bid_ref, ws2_ref, wss_ref,
             woh_ref, wout_ref, o_ref):
        i = pl.program_id(0)
        s_in = fin_ref[:, 0:16]
        sp = sp_ref[...]
        oh = (sp == lax.broadcasted_iota(
            jnp.int32, (_NB, NUM_SPECIES), 1).astype(jnp.float32)
              ).astype(jnp.float32)
        s2 = _dot(a_ref[...] * NORM, ws2_ref[...])
        for spc in range(NUM_SPECIES):
            s2 = s2 + oh[:, spc:spc + 1] * _dot(s_in, wss_ref[spc])
        s_new = s2 * _sigmoid(s2) + s_in
        e = _dot(_dot(s_new, woh_ref[...]), wout_ref[...])
        ohb = (bid_ref[...] == lax.broadcasted_iota(
            jnp.int32, (_NB, NUM_GRAPHS), 1).astype(jnp.float32)
               ).astype(jnp.float32)
        contrib = jnp.sum(ohb * e, axis=0, keepdims=True)

        @pl.when(i == 0)
        def _():
            o_ref[...] = jnp.zeros_like(o_ref)

        o_ref[...] += contrib

    out = pl.pallas_call(
        body, grid=(N // _NB,),
        in_specs=[pl.BlockSpec((_NB, 16), lambda i: (i, 0)),
                  pl.BlockSpec((_NB, 64), lambda i: (i, 0)),
                  pl.BlockSpec((_NB, 1), lambda i: (i, 0)),
                  pl.BlockSpec((_NB, 1), lambda i: (i, 0)),
                  pl.BlockSpec((MUL, MUL), lambda i: (0, 0)),
                  pl.BlockSpec((NUM_SPECIES, MUL, MUL), lambda i: (0, 0, 0)),
                  pl.BlockSpec((MUL, MUL), lambda i: (0, 0)),
                  pl.BlockSpec((MUL, 1), lambda i: (0, 0))],
        out_specs=pl.BlockSpec((1, NUM_GRAPHS), lambda i: (0, 0)),
        out_shape=jax.ShapeDtypeStruct((1, NUM_GRAPHS), jnp.float32),
    )(A, Fin, sp1, bid1, Ws2l, Wscsl, W_oh, W_out)
    return out


def kernel(pos, W_chem, W_rad1, W_rad2, Ws1, Wv1, Ws2, Wv2,
           Wsc_s, Wsc_v, Wg, W_oh, W_out, edge_index, species, batch_ids):
    f32 = jnp.float32
    ei = jnp.asarray(edge_index, jnp.int32)
    srci = ei[0]
    dsti = ei[1]
    sp1 = jnp.asarray(species, f32).reshape(N, 1)
    bid1 = jnp.asarray(batch_ids, f32).reshape(N, 1)
    pos8 = jnp.concatenate([pos, jnp.zeros((N, 5), f32)], axis=1)
    z64 = jnp.zeros((_ZROWS, 64), f32)
    z16 = jnp.zeros((_ZROWS, 16), f32)

    gather8 = _make_gather(8)
    gather64 = _make_gather(64)
    scatter64 = _make_scatter(64)
    scatter16 = _make_scatter(16)

    ps = gather8(pos8, srci)
    pd = gather8(pos8, dsti)
    geom = _tc_geom(ps, pd)

    F, F1 = _tc_init(sp1, W_chem, Ws1[0])
    for l in range(NLAYER):
        G = gather64(F1, srci)
        if l < NLAYER - 1:
            M = _tc_edge(geom, G, W_rad1[l], W_rad2[l])
            A = scatter64(M, dsti, z64)
            F, F1 = _tc_post(A, F, sp1, Ws2[l], Wv2[l], Wsc_s[l],
                             Wsc_v[l], Wg[l], Ws1[l + 1], Wv1[l + 1])
        else:
            W2s = jnp.concatenate(
                [W_rad2[l][:, 0:16], W_rad2[l][:, 48:64]], axis=1)
            M = _tc_edge_s(geom, G, W_rad1[l], W2s)
            A = scatter16(M, dsti, z16)
            out = _tc_head(A, F, sp1, bid1, Ws2[l], Wsc_s[l], W_oh, W_out)
    return out.reshape(NUM_GRAPHS, 1)
